# Initial kernel scaffold; baseline (speedup 1.0000x reference)
import jax
import jax.numpy as jnp
from jax.experimental import pallas as pl


def kernel(z_e, embeddings):
    raise NotImplementedError("write your pallas kernel here")


# speedup vs baseline: 1.1059x; 1.1059x over previous
import functools

import jax
import jax.numpy as jnp
from jax import lax
from jax.experimental import pallas as pl
from jax.experimental.pallas import tpu as pltpu
from jax.experimental.pallas import tpu_sc as plsc

N_EMBED = 8192
D_MODEL = 64
BETA = 0.25
N_TOK = 16 * 1024

RB = 256
CB = 2048

NW = 32
BPW = N_TOK // NW
IDXC = 128
LANES = 16


def _tc_argmin_body(z_ref, e_ref, s1_ref, se_ref, out_ref):
    z = z_ref[...]
    s1 = s1_ref[...]
    run_m = jnp.full((RB, 1), jnp.inf, jnp.float32)
    run_i = jnp.zeros((RB, 1), jnp.int32)
    for c in range(N_EMBED // CB):
        ec = e_ref[c * CB:(c + 1) * CB, :]
        se = se_ref[:, c * CB:(c + 1) * CB]
        t = lax.dot_general(z, ec, (((1,), (1,)), ((), ())),
                            preferred_element_type=jnp.float32)
        sq = (s1 - 2.0 * t) + se
        cc = jnp.maximum(sq, 0.0)
        dist = cc * lax.rsqrt(cc)
        m = jnp.min(dist, axis=1, keepdims=True)
        iota = lax.broadcasted_iota(jnp.int32, (RB, CB), 1) + c * CB
        cand = jnp.where(dist == m, iota, jnp.int32(2 ** 30))
        li = jnp.min(cand, axis=1, keepdims=True)
        better = m < run_m
        run_i = jnp.where(better, li, run_i)
        run_m = jnp.where(better, m, run_m)
    out_ref[...] = run_i


_tc_argmin = pl.pallas_call(
    _tc_argmin_body,
    grid=(N_TOK // RB,),
    in_specs=[
        pl.BlockSpec((RB, D_MODEL), lambda i: (i, 0)),
        pl.BlockSpec((N_EMBED, D_MODEL), lambda i: (0, 0)),
        pl.BlockSpec((RB, 1), lambda i: (i, 0)),
        pl.BlockSpec((1, N_EMBED), lambda i: (0, 0)),
    ],
    out_specs=pl.BlockSpec((RB, 1), lambda i: (i, 0)),
    out_shape=jax.ShapeDtypeStruct((N_TOK, 1), jnp.int32),
)


def _sc_gather_body(emb_hbm, idx_hbm, ze_hbm, zq_hbm, part_hbm,
                    idx_v, zq_v, ze_v, acc_v, sem):
    wid = lax.axis_index("s") * 2 + lax.axis_index("c")
    base = wid * BPW
    nch = BPW // IDXC
    pltpu.sync_copy(idx_hbm.at[pl.ds(wid * nch, nch)], idx_v)
    cps = [
        pltpu.async_copy(emb_hbm.at[idx_v.at[j]],
                         zq_v.at[pl.ds(j * IDXC, IDXC)], sem)
        for j in range(nch)
    ]
    for cp in cps:
        cp.wait()
    pltpu.sync_copy(ze_hbm.at[pl.ds(base, BPW)], ze_v)

    def body(i, acc):
        for j in range(D_MODEL // LANES):
            sl = pl.ds(j * LANES, LANES)
            zq = zq_v[i, sl]
            ze = ze_v[i, sl]
            d = zq - ze
            acc = acc + d * d
            zq_v[i, sl] = ze + d
        return acc

    acc = lax.fori_loop(0, BPW, body, jnp.zeros((LANES,), jnp.float32))
    acc_v[...] = acc
    pltpu.sync_copy(zq_v, zq_hbm.at[pl.ds(base, BPW)])
    pltpu.sync_copy(acc_v, part_hbm.at[wid])


@functools.cache
def _make_sc_gather():
    return pl.kernel(
        _sc_gather_body,
        mesh=plsc.VectorSubcoreMesh(core_axis_name="c", subcore_axis_name="s",
                                    num_cores=2, num_subcores=16),
        compiler_params=pltpu.CompilerParams(use_tc_tiling_on_sc=False),
        out_type=[
            jax.ShapeDtypeStruct((N_TOK, D_MODEL), jnp.float32),
            jax.ShapeDtypeStruct((NW, LANES), jnp.float32),
        ],
        scratch_types=[
            pltpu.VMEM((BPW // IDXC, IDXC), jnp.int32),
            pltpu.VMEM((BPW, D_MODEL), jnp.float32),
            pltpu.VMEM((BPW, D_MODEL), jnp.float32),
            pltpu.VMEM((LANES,), jnp.float32),
            pltpu.SemaphoreType.DMA,
        ],
    )


def kernel(z_e, embeddings):
    z_flat = z_e.reshape(-1, D_MODEL)
    s1 = jnp.sum(z_flat ** 2, axis=1, keepdims=True)
    se = jnp.sum(embeddings ** 2, axis=1)[None, :]
    idx = _tc_argmin(z_flat, embeddings, s1, se)
    idx_flat = idx.reshape(-1)
    idx2d = idx_flat.reshape(-1, IDXC)
    zq_st, part = _make_sc_gather()(embeddings, idx2d, z_flat)
    m = jnp.sum(part) / (N_TOK * D_MODEL)
    loss = BETA * m + m
    return (zq_st.reshape(z_e.shape), loss,
            idx_flat.reshape(z_e.shape[:-1]))
